# baseline (speedup 1.0000x reference)
import functools

import jax
import jax.numpy as jnp
from jax import lax
from jax.experimental import pallas as pl
from jax.experimental.pallas import tpu as pltpu
from jax.experimental.pallas import tpu_sc as plsc

N_USERS = 5000
N_NODES = 10000
NPAD = 10240
DH = 64
NE = 320000
CHUNK = 128
GSZ = 18
NG = 9
NTOT = GSZ * NG
NIT = NTOT // 3
EPAD = 16 * NTOT * CHUNK
NSL = NPAD // 16
DUMMY = N_NODES
NLAYERS = 3
CB = CHUNK * DH * 4
DB = CHUNK * 16 * 4

_mesh = plsc.VectorSubcoreMesh(
    core_axis_name="c", subcore_axis_name="s", num_cores=2, num_subcores=16
)


@functools.partial(
    pl.kernel,
    out_type=[
        jax.ShapeDtypeStruct((2, NPAD, DH), jnp.float32),
        jax.ShapeDtypeStruct((2 * NPAD, DH), jnp.float32),
        jax.ShapeDtypeStruct((GSZ * CHUNK, 16), jnp.float32),
    ],
    mesh=_mesh,
    scratch_types=[
        pltpu.VMEM((2, GSZ, CHUNK), jnp.int32),
        pltpu.VMEM((2, GSZ, CHUNK), jnp.int32),
        pltpu.VMEM((CHUNK, DH), jnp.float32),
        pltpu.VMEM((CHUNK, DH), jnp.float32),
        pltpu.VMEM((CHUNK, DH), jnp.float32),
        pltpu.VMEM((CHUNK, DH), jnp.float32),
        pltpu.VMEM((CHUNK, DH), jnp.float32),
        pltpu.VMEM((NSL, 16), jnp.float32),
        pltpu.VMEM((CHUNK, 16), jnp.float32),
        pltpu.VMEM_SHARED((NPAD, DH), jnp.float32),
        pltpu.VMEM_SHARED((NPAD, 16), jnp.float32),
        pltpu.SemaphoreType.DMA,
        pltpu.SemaphoreType.DMA,
        pltpu.SemaphoreType.DMA,
        pltpu.SemaphoreType.DMA,
        pltpu.SemaphoreType.DMA,
        pltpu.SemaphoreType.DMA,
        pltpu.SemaphoreType.DMA,
    ],
    compiler_params=pltpu.CompilerParams(use_tc_tiling_on_sc=False),
)
def _lightgcn_sc(xin, rows_h, cols_h, out, ybuf, ddum,
                 rowsb, colsb, gbuf0, gbuf1, gbuf2, wb, wb2, dv, onesb,
                 acc, degs, semg0, semg1, semg2, sems0, sems1, sems2, semd):
    cid = lax.axis_index("c")
    sid = lax.axis_index("s")
    base_n = sid * NSL
    xoff = cid * NPAD + base_n
    off = (cid * NPAD).astype(jnp.int32)
    gbufs = [gbuf0, gbuf1, gbuf2]
    semg = [semg0, semg1, semg2]
    sems = [sems0, sems1, sems2]

    def _drain_gather(b):
        pltpu.make_async_copy(ybuf.at[pl.ds(0, CHUNK)], gbufs[b],
                              semg[b]).wait()

    def _drain_scatter(k):
        pltpu.make_async_copy(ybuf.at[pl.ds(0, CHUNK)],
                              acc.at[pl.ds(0, CHUNK)], sems[k]).wait()

    def _drain_deg():
        pltpu.make_async_copy(ddum, degs.at[pl.ds(0, GSZ * CHUNK)],
                              semd).wait()

    def _fill_ones(j, _):
        onesb[j, :] = jnp.full((16,), 1.0, jnp.float32)
        return 0
    lax.fori_loop(0, CHUNK, _fill_ones, 0)

    def _zero_wb(j, _):
        for k in range(DH // 16):
            wb[j, pl.ds(k * 16, 16)] = jnp.zeros((16,), jnp.float32)
        return 0

    def _zero_dv(n, _):
        dv[n, :] = jnp.zeros((16,), jnp.float32)
        return 0
    lax.fori_loop(0, NSL, _zero_dv, 0)
    pltpu.sync_copy(dv, degs.at[pl.ds(base_n, NSL)])
    lax.fori_loop(0, CHUNK, _zero_wb, 0)
    for t in range(NSL // CHUNK):
        pltpu.sync_copy(wb, acc.at[pl.ds(base_n + t * CHUNK, CHUNK)])
    plsc.subcore_barrier()

    def _load_cols(g, slot):
        pltpu.sync_copy(cols_h.at[sid, g], colsb.at[slot])

    def _load_rows(g, slot):
        pltpu.sync_copy(rows_h.at[sid, g], rowsb.at[slot])

        def _shift(j, _):
            for k in range(CHUNK // 16):
                sl = pl.ds(k * 16, 16)
                rowsb[slot, j, sl] = rowsb[slot, j, sl] + off
            return 0
        lax.fori_loop(0, GSZ, _shift, 0)

    def _deg_group(g, _):
        _load_cols(g, 0)

        def _deg_chunk(j, _):
            pltpu.sync_copy(onesb, degs.at[colsb.at[0, j]], add=True)
            return 0
        lax.fori_loop(0, GSZ, _deg_chunk, 0)
        return 0
    lax.fori_loop(0, NG, _deg_group, 0)
    plsc.subcore_barrier()

    pltpu.sync_copy(degs.at[pl.ds(base_n, NSL)], dv)

    def _newton(n, _):
        d = dv[n, :]
        i = lax.bitcast_convert_type(d, jnp.int32)
        y = lax.bitcast_convert_type(
            jnp.full((16,), 0x5F3759DF, jnp.int32) - (i >> 1), jnp.float32)
        for _ in range(3):
            y = y * (1.5 - 0.5 * d * y * y)
        dv[n, :] = jnp.where(d > 0.5, y, jnp.zeros((16,), jnp.float32))
        return 0
    lax.fori_loop(0, NSL, _newton, 0)

    for t in range(NSL // CHUNK):
        pltpu.sync_copy(xin.at[pl.ds(xoff + t * CHUNK, CHUNK)], wb)

        def _y0(m, _, t=t):
            n = t * CHUNK + m
            b = dv[n, :]
            for k in range(DH // 16):
                sl = pl.ds(k * 16, 16)
                a = wb[m, sl]
                wb2[m, sl] = a
                wb[m, sl] = a * b
            return 0
        lax.fori_loop(0, CHUNK, _y0, 0)
        pltpu.sync_copy(wb2, out.at[cid, pl.ds(base_n + t * CHUNK, CHUNK)])
        pltpu.sync_copy(wb, ybuf.at[pl.ds(xoff + t * CHUNK, CHUNK)])
    plsc.subcore_barrier()

    for layer in range(NLAYERS):
        last = layer == NLAYERS - 1

        def _fire_gather_l(c, b):
            jm = c % GSZ
            g = c // GSZ
            slot = g & 1

            @pl.when(jm == 0)
            def _():
                _load_rows(g, slot)
                _load_cols(g, slot)
            pltpu.async_copy(ybuf.at[rowsb.at[slot, jm]], gbufs[b], semg[b])

        _load_rows(0, 0)
        _load_cols(0, 0)
        pltpu.async_copy(ybuf.at[rowsb.at[0, 0]], gbufs[0], semg[0])
        pltpu.async_copy(ybuf.at[rowsb.at[0, 1]], gbufs[1], semg[1])

        def _pipe(i, _):
            for k in range(3):
                c = 3 * i + k
                b2 = (k + 2) % 3
                _drain_gather(k)
                jm = c % GSZ
                slot = (c // GSZ) & 1
                pltpu.sync_copy(gbufs[k], acc.at[colsb.at[slot, jm]],
                                add=True)
                if k == 0:
                    _fire_gather_l(c + 2, b2)
                else:
                    @pl.when(i < NIT - 1)
                    def _(c=c, b2=b2):
                        _fire_gather_l(c + 2, b2)
            return 0
        lax.fori_loop(0, NIT, _pipe, 0)
        plsc.subcore_barrier()

        for t in range(NSL // CHUNK):
            sl_nodes = pl.ds(base_n + t * CHUNK, CHUNK)
            out_sl = out.at[cid, pl.ds(base_n + t * CHUNK, CHUNK)]
            pltpu.sync_copy(acc.at[sl_nodes], wb)
            pltpu.sync_copy(out_sl, wb2)

            def _nodes(m, _, t=t, last=last):
                n = t * CHUNK + m
                b = dv[n, :]
                for k in range(DH // 16):
                    sl = pl.ds(k * 16, 16)
                    a = wb[m, sl] * b
                    s = wb2[m, sl] + a
                    if last:
                        wb2[m, sl] = s * (1.0 / (NLAYERS + 1))
                    else:
                        wb2[m, sl] = s
                        wb[m, sl] = a * b
                return 0
            lax.fori_loop(0, CHUNK, _nodes, 0)
            pltpu.sync_copy(wb2, out_sl)
            if not last:
                pltpu.sync_copy(wb, ybuf.at[pl.ds(xoff + t * CHUNK, CHUNK)])
                lax.fori_loop(0, CHUNK, _zero_wb, 0)
                pltpu.sync_copy(wb, acc.at[sl_nodes])
        plsc.subcore_barrier()


@jax.jit
def kernel(user_emb, item_emb, edge_index):
    x = jnp.concatenate([user_emb, item_emb], axis=0)
    xpad = jnp.pad(x, ((0, NPAD - N_NODES), (0, 0)))
    xin = jnp.concatenate([xpad[:, :DH], xpad[:, DH:]], axis=0)

    rows = edge_index[0].astype(jnp.int32)
    cols = edge_index[1].astype(jnp.int32)
    rows = jnp.pad(rows, (0, EPAD - NE), constant_values=DUMMY)
    cols = jnp.pad(cols, (0, EPAD - NE), constant_values=DUMMY)
    rows_h = rows.reshape(16, NG, GSZ, CHUNK)
    cols_h = cols.reshape(16, NG, GSZ, CHUNK)

    out, _y, _d = _lightgcn_sc(xin, rows_h, cols_h)
    final = jnp.concatenate([out[0, :N_NODES], out[1, :N_NODES]], axis=1)
    return final[:N_USERS], final[N_USERS:]
